# Initial kernel scaffold; baseline (speedup 1.0000x reference)
import jax
import jax.numpy as jnp
from jax.experimental import pallas as pl


def kernel(X, W1, b1, gamma1, beta1, W2, b2, gamma2, beta2):
    raise NotImplementedError("write your pallas kernel here")


# speedup vs baseline: 8.4689x; 8.4689x over previous
import functools

import jax
import jax.numpy as jnp
from jax import lax
from jax.experimental import pallas as pl
from jax.experimental.pallas import tpu as pltpu
from jax.experimental.pallas import tpu_sc as plsc

K = 20
B, N, F = 8, 2048, 3
C1, C2 = 64, 64
BNK = B * N * K
DPAD = 128
RBLK = 256
EPS = 1e-5


def _knn_body(x_ref, xt_ref, idx_ref):
    b = pl.program_id(0)
    rb = pl.program_id(1)
    x = x_ref[0]
    xt = xt_ref[0]
    dot = lax.dot_general(x, xt, (((1,), (0,)), ((), ())),
                          preferred_element_type=jnp.float32)
    sq_r = jnp.sum(x * x, axis=1, keepdims=True)
    sq_c = jnp.sum(xt * xt, axis=0, keepdims=True)
    dist = sq_r + sq_c - 2.0 * dot
    lane = lax.broadcasted_iota(jnp.int32, (RBLK, N), 1)
    inf = jnp.float32(jnp.inf)
    cols = []
    for k in range(K + 1):
        m = jnp.min(dist, axis=1, keepdims=True)
        cand = jnp.where(dist <= m, lane, N)
        sel = jnp.min(cand, axis=1, keepdims=True)
        if k > 0:
            cols.append(sel)
        dist = jnp.where(lane == sel, inf, dist)
    idx_ref[0] = jnp.concatenate(cols, axis=1) + b * N


def _knn_indices(x, xt):
    return pl.pallas_call(
        _knn_body,
        grid=(B, N // RBLK),
        in_specs=[
            pl.BlockSpec((1, RBLK, F), lambda b, rb: (b, rb, 0)),
            pl.BlockSpec((1, F, N), lambda b, rb: (b, 0, 0)),
        ],
        out_specs=pl.BlockSpec((1, RBLK, K), lambda b, rb: (b, rb, 0)),
        out_shape=jax.ShapeDtypeStruct((B, N, K), jnp.int32),
    )(x, xt)


_NC, _NS = 2, 16
_NW = _NC * _NS
_IDX_ROWS = BNK // 128
_ROWS_PER_W = _IDX_ROWS // _NW
_CHUNK_IR = 4
_NCHUNK = _ROWS_PER_W // _CHUNK_IR
_CHUNK_E = _CHUNK_IR * 128


@functools.cache
def _sc_gather_fn():
    @functools.partial(
        pl.kernel,
        out_type=jax.ShapeDtypeStruct((BNK, DPAD), jnp.float32),
        mesh=plsc.VectorSubcoreMesh(core_axis_name="c", subcore_axis_name="s",
                                    num_cores=_NC, num_subcores=_NS),
        scratch_types=[
            pltpu.VMEM((_CHUNK_IR, 128), jnp.int32),
            pltpu.VMEM((_CHUNK_E, DPAD), jnp.float32),
            pltpu.SemaphoreType.DMA,
        ],
    )
    def sc_gather(table_hbm, idx_hbm, out_hbm, idx_v, rows_v, sem):
        wid = lax.axis_index("s") * _NC + lax.axis_index("c")

        def chunk(ci, carry):
            row0 = wid * _ROWS_PER_W + ci * _CHUNK_IR
            pltpu.sync_copy(idx_hbm.at[pl.ds(row0, _CHUNK_IR)], idx_v)
            cps = []
            for j in range(_CHUNK_IR):
                cps.append(pltpu.async_copy(
                    table_hbm.at[idx_v.at[j]],
                    rows_v.at[pl.ds(j * 128, 128)], sem))
            for cp in cps:
                cp.wait()
            ebase = (wid * _NCHUNK + ci) * _CHUNK_E
            pltpu.sync_copy(rows_v, out_hbm.at[pl.ds(ebase, _CHUNK_E)])
            return carry

        lax.fori_loop(0, _NCHUNK, chunk, 0)

    return sc_gather


def _sc_gather(table, idx2d):
    return _sc_gather_fn()(table, idx2d)


NB = 256


def _stats1_body(x_ref, xj_ref, wu_ref, wv_ref, b1_ref, s_ref):
    first = (pl.program_id(0) == 0) & (pl.program_id(1) == 0)

    @pl.when(first)
    def _():
        s_ref[...] = jnp.zeros_like(s_ref)

    x = x_ref[0]
    u = lax.dot_general(x, wu_ref[...], (((1,), (0,)), ((), ())),
                        preferred_element_type=jnp.float32,
                        precision=lax.Precision.HIGHEST) + b1_ref[...]
    s = jnp.zeros((1, C1), jnp.float32)
    ssq = jnp.zeros((1, C1), jnp.float32)
    for k in range(K):
        xk = xj_ref[0][:, k * DPAD:(k + 1) * DPAD]
        v = lax.dot_general(xk, wv_ref[...], (((1,), (0,)), ((), ())),
                            preferred_element_type=jnp.float32,
                            precision=lax.Precision.HIGHEST)
        y = u + v
        s = s + jnp.sum(y, axis=0, keepdims=True)
        ssq = ssq + jnp.sum(y * y, axis=0, keepdims=True)
    s_ref[0:1, :C1] += s
    s_ref[1:2, :C1] += ssq


def _stats1(x, xj, wu, wv, b1r):
    return pl.pallas_call(
        _stats1_body,
        grid=(B, N // NB),
        in_specs=[
            pl.BlockSpec((1, NB, F), lambda b, nb: (b, nb, 0)),
            pl.BlockSpec((1, NB, K * DPAD), lambda b, nb: (b, nb, 0)),
            pl.BlockSpec((F, C1), lambda b, nb: (0, 0)),
            pl.BlockSpec((DPAD, C1), lambda b, nb: (0, 0)),
            pl.BlockSpec((1, C1), lambda b, nb: (0, 0)),
        ],
        out_specs=pl.BlockSpec((8, 128), lambda b, nb: (0, 0)),
        out_shape=jax.ShapeDtypeStruct((8, 128), jnp.float32),
    )(x, xj, wu, wv, b1r)


def _fused_body(x_ref, xj_ref, st1_ref, wu_ref, wv_ref, b1_ref, g1_ref,
                bt1_ref, w2t_ref, b2_ref, mx_ref, mn_ref, s2_ref):
    first = (pl.program_id(0) == 0) & (pl.program_id(1) == 0)

    @pl.when(first)
    def _():
        s2_ref[...] = jnp.zeros_like(s2_ref)

    cnt = jnp.float32(BNK)
    mean1 = st1_ref[0:1, :C1] / cnt
    var1 = st1_ref[1:2, :C1] / cnt - mean1 * mean1
    rstd1 = 1.0 / jnp.sqrt(var1 + EPS)
    a1 = g1_ref[...] * rstd1
    c1 = bt1_ref[...] - mean1 * a1

    x = x_ref[0]
    u = lax.dot_general(x, wu_ref[...], (((1,), (0,)), ((), ())),
                        preferred_element_type=jnp.float32,
                        precision=lax.Precision.HIGHEST) + b1_ref[...]
    mx = jnp.full((NB, C2), -jnp.inf, jnp.float32)
    mn = jnp.full((NB, C2), jnp.inf, jnp.float32)
    s2 = jnp.zeros((1, C2), jnp.float32)
    ssq2 = jnp.zeros((1, C2), jnp.float32)
    for k in range(K):
        xk = xj_ref[0][:, k * DPAD:(k + 1) * DPAD]
        v = lax.dot_general(xk, wv_ref[...], (((1,), (0,)), ((), ())),
                            preferred_element_type=jnp.float32,
                            precision=lax.Precision.HIGHEST)
        e = jnp.maximum(a1 * (u + v) + c1, 0.0)
        y2 = lax.dot_general(e, w2t_ref[...], (((1,), (0,)), ((), ())),
                             preferred_element_type=jnp.float32,
                             precision=lax.Precision.HIGHEST) + b2_ref[...]
        mx = jnp.maximum(mx, y2)
        mn = jnp.minimum(mn, y2)
        s2 = s2 + jnp.sum(y2, axis=0, keepdims=True)
        ssq2 = ssq2 + jnp.sum(y2 * y2, axis=0, keepdims=True)
    mx_ref[0] = mx
    mn_ref[0] = mn
    s2_ref[0:1, :C2] += s2
    s2_ref[1:2, :C2] += ssq2


def _fused(x, xj, st1, wu, wv, b1r, g1r, bt1r, w2t, b2r):
    return pl.pallas_call(
        _fused_body,
        grid=(B, N // NB),
        in_specs=[
            pl.BlockSpec((1, NB, F), lambda b, nb: (b, nb, 0)),
            pl.BlockSpec((1, NB, K * DPAD), lambda b, nb: (b, nb, 0)),
            pl.BlockSpec((8, 128), lambda b, nb: (0, 0)),
            pl.BlockSpec((F, C1), lambda b, nb: (0, 0)),
            pl.BlockSpec((DPAD, C1), lambda b, nb: (0, 0)),
            pl.BlockSpec((1, C1), lambda b, nb: (0, 0)),
            pl.BlockSpec((1, C1), lambda b, nb: (0, 0)),
            pl.BlockSpec((1, C1), lambda b, nb: (0, 0)),
            pl.BlockSpec((C1, C2), lambda b, nb: (0, 0)),
            pl.BlockSpec((1, C2), lambda b, nb: (0, 0)),
        ],
        out_specs=[
            pl.BlockSpec((1, NB, C2), lambda b, nb: (b, nb, 0)),
            pl.BlockSpec((1, NB, C2), lambda b, nb: (b, nb, 0)),
            pl.BlockSpec((8, 128), lambda b, nb: (0, 0)),
        ],
        out_shape=[
            jax.ShapeDtypeStruct((B, N, C2), jnp.float32),
            jax.ShapeDtypeStruct((B, N, C2), jnp.float32),
            jax.ShapeDtypeStruct((8, 128), jnp.float32),
        ],
    )(x, xj, st1, wu, wv, b1r, g1r, bt1r, w2t, b2r)


def _finish_body(mx_ref, mn_ref, s2_ref, g2_ref, bt2_ref, o_ref):
    cnt = jnp.float32(BNK)
    mean2 = s2_ref[0:1, :C2] / cnt
    var2 = s2_ref[1:2, :C2] / cnt - mean2 * mean2
    rstd2 = 1.0 / jnp.sqrt(var2 + EPS)
    a2 = g2_ref[...] * rstd2
    c2 = bt2_ref[...] - mean2 * a2
    t = jnp.where(a2 >= 0.0, mx_ref[0], mn_ref[0])
    o_ref[0] = jnp.maximum(a2 * t + c2, 0.0)


def _finish(mx, mn, s2, g2r, bt2r):
    return pl.pallas_call(
        _finish_body,
        grid=(B,),
        in_specs=[
            pl.BlockSpec((1, N, C2), lambda b: (b, 0, 0)),
            pl.BlockSpec((1, N, C2), lambda b: (b, 0, 0)),
            pl.BlockSpec((8, 128), lambda b: (0, 0)),
            pl.BlockSpec((1, C2), lambda b: (0, 0)),
            pl.BlockSpec((1, C2), lambda b: (0, 0)),
        ],
        out_specs=pl.BlockSpec((1, N, C2), lambda b: (b, 0, 0)),
        out_shape=jax.ShapeDtypeStruct((B, N, C2), jnp.float32),
    )(mx, mn, s2, g2r, bt2r)


def kernel(X, W1, b1, gamma1, beta1, W2, b2, gamma2, beta2):
    X = X.astype(jnp.float32)
    xt = jnp.transpose(X, (0, 2, 1))

    idx = _knn_indices(X, xt)
    idx2d = idx.reshape(_IDX_ROWS, 128)

    table = jnp.zeros((B * N, DPAD), jnp.float32)
    table = table.at[:, :F].set(X.reshape(B * N, F))
    xj = _sc_gather(table, idx2d).reshape(B, N, K * DPAD)

    wu = (W1[:, :F] - W1[:, F:]).T
    wv = jnp.zeros((DPAD, C1), jnp.float32).at[:F].set(W1[:, F:].T)
    w2t = W2.T
    b1r, g1r, bt1r = b1[None, :], gamma1[None, :], beta1[None, :]
    b2r, g2r, bt2r = b2[None, :], gamma2[None, :], beta2[None, :]

    st1 = _stats1(X, xj, wu, wv, b1r)
    mx, mn, st2 = _fused(X, xj, st1, wu, wv, b1r, g1r, bt1r, w2t, b2r)
    return _finish(mx, mn, st2, g2r, bt2r)
